# baseline (speedup 1.0000x reference)
import functools

import jax
import jax.numpy as jnp
from jax import lax
from jax.experimental import pallas as pl
from jax.experimental.pallas import tpu as pltpu
from jax.experimental.pallas import tpu_sc as plsc

MAX_LEN = 100000
D_MODEL = 64
BATCH = 16384

_info = plsc.get_sparse_core_info()
_NC, _NS = _info.num_cores, _info.num_subcores
_NW = _NC * _NS
_COLS_PER_W = D_MODEL // _NW
_CHUNK = 8192


def _gather_body(table_t_hbm, idx_hbm, out_t_hbm, col_v, idx_v, out_v, sem_idx, sem_col):
    wid = lax.axis_index("s") * _NC + lax.axis_index("c")
    idx_cp = pltpu.async_copy(idx_hbm, idx_v, sem_idx)
    col_cp = pltpu.async_copy(table_t_hbm.at[wid * _COLS_PER_W], col_v, sem_col)
    idx_cp.wait()
    for ci in range(_COLS_PER_W):
        c = wid * _COLS_PER_W + ci
        col_cp.wait()
        nchunks = BATCH // _CHUNK
        for ki in range(nchunks):
            off = ki * _CHUNK

            @plsc.parallel_loop(0, _CHUNK, step=16, unroll=16)
            def _gather16(j):
                iv = idx_v[pl.ds(off + j, 16)]
                out_v[pl.ds(j, 16)] = plsc.load_gather(col_v, [iv])

            if ki == nchunks - 1 and ci + 1 < _COLS_PER_W:
                col_cp = pltpu.async_copy(table_t_hbm.at[c + 1], col_v, sem_col)
            pltpu.sync_copy(out_v, out_t_hbm.at[c, pl.ds(off, _CHUNK)])


@jax.jit
def _sc_gather(table_t, idx):
    mesh = plsc.VectorSubcoreMesh(core_axis_name="c", subcore_axis_name="s")
    return pl.kernel(
        _gather_body,
        mesh=mesh,
        out_type=jax.ShapeDtypeStruct((D_MODEL, BATCH), jnp.float32),
        scratch_types=[
            pltpu.VMEM((MAX_LEN,), jnp.float32),
            pltpu.VMEM((BATCH,), jnp.int32),
            pltpu.VMEM((_CHUNK,), jnp.float32),
            pltpu.SemaphoreType.DMA,
            pltpu.SemaphoreType.DMA,
        ],
        compiler_params=pltpu.CompilerParams(
            use_tc_tiling_on_sc=True, needs_layout_passes=False
        ),
    )(table_t, idx)


def kernel(window_position, window_size, pos_embedding):
    del window_size
    out_t = _sc_gather(pos_embedding.T, window_position.astype(jnp.int32))
    return out_t.T


# speedup vs baseline: 1.0090x; 1.0090x over previous
import functools

import jax
import jax.numpy as jnp
from jax import lax
from jax.experimental import pallas as pl
from jax.experimental.pallas import tpu as pltpu
from jax.experimental.pallas import tpu_sc as plsc

MAX_LEN = 100000
D_MODEL = 64
BATCH = 16384

_info = plsc.get_sparse_core_info()
_NC, _NS = _info.num_cores, _info.num_subcores
_NW = _NC * _NS
_COLS_PER_W = D_MODEL // _NW
_CHUNK = 8192


def _gather_body(table_t_hbm, idx_hbm, out_t_hbm, col_v, idx_v, out_v, sem_idx, sem_col):
    wid = lax.axis_index("s") * _NC + lax.axis_index("c")
    idx_cp = pltpu.async_copy(idx_hbm, idx_v, sem_idx)
    col_cp = pltpu.async_copy(table_t_hbm.at[wid * _COLS_PER_W], col_v, sem_col)
    idx_cp.wait()
    for ci in range(_COLS_PER_W):
        c = wid * _COLS_PER_W + ci
        col_cp.wait()
        nchunks = BATCH // _CHUNK
        for ki in range(nchunks):
            off = ki * _CHUNK

            @plsc.parallel_loop(0, _CHUNK, step=16, unroll=8)
            def _gather16(j):
                iv = idx_v[pl.ds(off + j, 16)]
                out_v[pl.ds(j, 16)] = plsc.load_gather(col_v, [iv])

            if ki == nchunks - 1 and ci + 1 < _COLS_PER_W:
                col_cp = pltpu.async_copy(table_t_hbm.at[c + 1], col_v, sem_col)
            pltpu.sync_copy(out_v, out_t_hbm.at[c, pl.ds(off, _CHUNK)])


@jax.jit
def _sc_gather(table_t, idx):
    mesh = plsc.VectorSubcoreMesh(core_axis_name="c", subcore_axis_name="s")
    return pl.kernel(
        _gather_body,
        mesh=mesh,
        out_type=jax.ShapeDtypeStruct((D_MODEL, BATCH), jnp.float32),
        scratch_types=[
            pltpu.VMEM((MAX_LEN,), jnp.float32),
            pltpu.VMEM((BATCH,), jnp.int32),
            pltpu.VMEM((_CHUNK,), jnp.float32),
            pltpu.SemaphoreType.DMA,
            pltpu.SemaphoreType.DMA,
        ],
        compiler_params=pltpu.CompilerParams(
            use_tc_tiling_on_sc=True, needs_layout_passes=False
        ),
    )(table_t, idx)


def kernel(window_position, window_size, pos_embedding):
    del window_size
    out_t = _sc_gather(pos_embedding.T, window_position.astype(jnp.int32))
    return out_t.T
